# baseline (speedup 1.0000x reference)
import functools

import jax
import jax.numpy as jnp
from jax import lax
from jax.experimental import pallas as pl
from jax.experimental.pallas import tpu as pltpu
from jax.experimental.pallas import tpu_sc as plsc

_CH = 128
_M = 32
_N = _CH * _M
_K = 64
_NW = 32
_LG2N = 13
_G = 8


def _thresh(v):
    return jnp.where(v < 0, 0.0, v)


def _tc_scan_body(x_ref, c8_ref, f8_ref, tot_ref, xt_ref, c_ref):
    RB = x_ref.shape[0]

    for m in range(_M):
        xt_ref[:, m, :] = jnp.transpose(x_ref[:, m, :])

    carry = _thresh(xt_ref[0])
    c_ref[0] = carry
    for jj in range(1, _CH):
        carry = carry + _thresh(xt_ref[jj])
        c_ref[jj] = carry

    T = c_ref[_CH - 1]
    pm = jnp.zeros((RB,), jnp.float32)
    plist = []
    for m in range(_M):
        plist.append(pm)
        pm = pm + T[m]
    P = jnp.stack(plist, axis=0)

    c_ref[...] = c_ref[...] + P[None, :, :]
    tot_ref[...] = pm[None, :]

    for m in range(_M):
        c8_ref[:, m, :, :] = jnp.transpose(c_ref[:, m, :]).reshape(
            RB // _G, _G, _CH)
        f8_ref[:, m, :, :] = jnp.transpose(_thresh(xt_ref[:, m, :])).reshape(
            RB // _G, _G, _CH)


def _sc_search_body(rpw, c8_hbm, u_hbm, tot_hbm, f8_hbm, s_hbm, p_hbm,
                    cbuf, fbuf, ubuf, tbuf, sall, pall, semc, semf):
    ng = rpw // _G
    wid = lax.axis_index("s") * 2 + lax.axis_index("c")
    base = wid * rpw
    gbase = wid * ng

    pltpu.sync_copy(u_hbm.at[pl.ds(base * _K, rpw * _K)], ubuf)
    pltpu.sync_copy(tot_hbm.at[pl.ds(base, rpw)], tbuf)

    pltpu.async_copy(c8_hbm.at[gbase], cbuf.at[0], semc.at[0])

    def group_body(g, _):
        buf = lax.rem(g, 2)
        nbuf = 1 - buf
        @pl.when(g + 1 < ng)
        def _():
            pltpu.async_copy(c8_hbm.at[gbase + g + 1], cbuf.at[nbuf],
                             semc.at[nbuf])
        fcopy = pltpu.async_copy(f8_hbm.at[gbase + g], fbuf, semf)
        pltpu.make_async_copy(c8_hbm.at[gbase + g], cbuf.at[buf],
                              semc.at[buf]).wait()

        buf16 = jnp.full((16,), buf, jnp.int32)

        def row_body(rlo, _2):
            r = g * _G + rlo
            t = plsc.load_gather(tbuf, [jnp.full((16,), r, jnp.int32)])
            off = rlo * _CH
            for gk in range(_K // 16):
                uraw = ubuf[pl.ds(r * _K + gk * 16, 16)]
                u2 = uraw * t
                lo = jnp.zeros((16,), jnp.int32)
                hi = jnp.full((16,), _N, jnp.int32)
                for _step in range(_LG2N):
                    mid = jnp.minimum(jnp.right_shift(lo + hi, 1), _N - 1)
                    adr = ((mid >> 7) << 10) + off + (mid & 127)
                    v = plsc.load_gather(cbuf, [buf16, adr])
                    pred = v <= u2
                    lo = jnp.where(pred, mid + 1, lo)
                    hi = jnp.where(pred, hi, mid)
                s = jnp.minimum(lo, _N - 1)
                sall[pl.ds(r * _K + gk * 16, 16)] = s
            return 0

        lax.fori_loop(0, _G, row_body, 0, unroll=False)

        fcopy.wait()

        def prob_body(rlo, _2):
            r = g * _G + rlo
            off = rlo * _CH
            for gk in range(_K // 16):
                s = sall[pl.ds(r * _K + gk * 16, 16)]
                adr = ((s >> 7) << 10) + off + (s & 127)
                vals = plsc.load_gather(fbuf, [adr])
                pall[pl.ds(r * _K + gk * 16, 16)] = vals
            return 0

        lax.fori_loop(0, _G, prob_body, 0, unroll=False)
        return 0

    lax.fori_loop(0, ng, group_body, 0, unroll=False)

    pltpu.sync_copy(sall, s_hbm.at[pl.ds(base * _K, rpw * _K)])
    pltpu.sync_copy(pall, p_hbm.at[pl.ds(base * _K, rpw * _K)])


def _tc_sort_body(s_ref, p_ref, xs_ref, ys_ref):
    RB, K = s_ref.shape
    s = jnp.transpose(s_ref[...])
    p = jnp.transpose(p_ref[...])

    ki = lax.broadcasted_iota(jnp.int32, (K, 1), 0)
    rank = jnp.zeros(s.shape, jnp.int32)
    for kq in range(K):
        pq = p[kq][None, :]
        before = (pq > p) | ((pq == p) & (kq < ki))
        rank = rank + before.astype(jnp.int32)

    s_sorted = jnp.zeros(s.shape, jnp.int32)
    for kq in range(K):
        hit = rank[kq][None, :] == ki
        s_sorted = s_sorted + jnp.where(hit, s[kq][None, :], 0)

    xf = (s_sorted & 63).astype(jnp.float32)
    yf = (s_sorted >> 6).astype(jnp.float32)
    xs_ref[...] = (xf - 32.0) * 0.015625
    ys_ref[...] = (yf - 32.0) * 0.015625


def _run_tc_scan(x_rows):
    Rh = x_rows.shape[0]
    RB = 256 if Rh % 256 == 0 else Rh
    return pl.pallas_call(
        _tc_scan_body,
        grid=(Rh // RB,),
        in_specs=[pl.BlockSpec((RB, _M, _CH), lambda i: (i, 0, 0))],
        out_specs=[
            pl.BlockSpec((RB // _G, _M, _G, _CH), lambda i: (i, 0, 0, 0)),
            pl.BlockSpec((RB // _G, _M, _G, _CH), lambda i: (i, 0, 0, 0)),
            pl.BlockSpec((1, RB), lambda i: (0, i)),
        ],
        out_shape=[
            jax.ShapeDtypeStruct((Rh // _G, _M, _G, _CH), jnp.float32),
            jax.ShapeDtypeStruct((Rh // _G, _M, _G, _CH), jnp.float32),
            jax.ShapeDtypeStruct((1, Rh), jnp.float32),
        ],
        scratch_shapes=[
            pltpu.VMEM((_CH, _M, RB), jnp.float32),
            pltpu.VMEM((_CH, _M, RB), jnp.float32),
        ],
        compiler_params=pltpu.CompilerParams(
            dimension_semantics=("arbitrary",),
        ),
    )(x_rows)


def _run_sc(c8, f8, u_flat, tot_flat):
    Rh = tot_flat.shape[0]
    rpw = Rh // _NW
    mesh = plsc.VectorSubcoreMesh(core_axis_name="c", subcore_axis_name="s")
    sc = pl.kernel(
        functools.partial(_sc_search_body, rpw),
        out_type=[
            jax.ShapeDtypeStruct((Rh * _K,), jnp.int32),
            jax.ShapeDtypeStruct((Rh * _K,), jnp.float32),
        ],
        mesh=mesh,
        scratch_types=[
            pltpu.VMEM((2, _M * _G * _CH), jnp.float32),
            pltpu.VMEM((_M * _G * _CH,), jnp.float32),
            pltpu.VMEM((rpw * _K,), jnp.float32),
            pltpu.VMEM((rpw,), jnp.float32),
            pltpu.VMEM((rpw * _K,), jnp.int32),
            pltpu.VMEM((rpw * _K,), jnp.float32),
            pltpu.SemaphoreType.DMA((2,)),
            pltpu.SemaphoreType.DMA,
        ],
        compiler_params=pltpu.CompilerParams(needs_layout_passes=False),
    )
    return sc(
        c8.reshape(Rh // _G, _M * _G * _CH),
        u_flat,
        tot_flat,
        f8.reshape(Rh // _G, _M * _G * _CH),
    )


def _run_tc_sort(s2d, p2d):
    Rh = s2d.shape[0]
    RB = 256 if Rh % 256 == 0 else Rh
    return pl.pallas_call(
        _tc_sort_body,
        grid=(Rh // RB,),
        in_specs=[
            pl.BlockSpec((RB, _K), lambda i: (i, 0)),
            pl.BlockSpec((RB, _K), lambda i: (i, 0)),
        ],
        out_specs=[
            pl.BlockSpec((_K, RB), lambda i: (0, i)),
            pl.BlockSpec((_K, RB), lambda i: (0, i)),
        ],
        out_shape=[
            jax.ShapeDtypeStruct((_K, Rh), jnp.float32),
            jax.ShapeDtypeStruct((_K, Rh), jnp.float32),
        ],
        compiler_params=pltpu.CompilerParams(
            dimension_semantics=("arbitrary",),
        ),
    )(s2d, p2d)


_UCACHE = {}


def _u_table(R):
    if R not in _UCACHE:
        import numpy as _np
        with jax.ensure_compile_time_eval():
            _UCACHE[R] = _np.asarray(
                jax.random.uniform(jax.random.key(42), (R, _K),
                                   dtype=jnp.float32))
    return _UCACHE[R]


def kernel(heatmap, num_samples):
    b, j, w, h = heatmap.shape
    R = b * j

    u_raw = jnp.asarray(_u_table(R))
    x_all = heatmap.reshape(R, _M, _CH)

    if R == 4352:
        splits = [(0, 2048), (2048, 2304)]
    else:
        splits = [(0, R)]

    xs_parts, ys_parts = [], []
    sc_outs = []
    for off, ln in splits:
        c8, f8, tot = _run_tc_scan(lax.slice_in_dim(x_all, off, off + ln, 1, 0))
        sc_outs.append(_run_sc(
            c8, f8,
            lax.slice_in_dim(u_raw.reshape(R * _K), off * _K,
                             (off + ln) * _K, 1, 0),
            tot.reshape(ln)))
    for (off, ln), (s_fl, p_fl) in zip(splits, sc_outs):
        xs_h, ys_h = _run_tc_sort(s_fl.reshape(ln, _K), p_fl.reshape(ln, _K))
        xs_parts.append(xs_h)
        ys_parts.append(ys_h)

    xs = jnp.concatenate(xs_parts, axis=1) if len(xs_parts) > 1 else xs_parts[0]
    ys = jnp.concatenate(ys_parts, axis=1) if len(ys_parts) > 1 else ys_parts[0]

    xn = xs.reshape(_K, b, j)
    yn = ys.reshape(_K, b, j)
    out = jnp.stack((xn, yn), axis=-1).transpose(1, 0, 2, 3).reshape(
        b, _K, 2 * j)
    return out


# speedup vs baseline: 1.0138x; 1.0138x over previous
import functools

import jax
import jax.numpy as jnp
from jax import lax
from jax.experimental import pallas as pl
from jax.experimental.pallas import tpu as pltpu
from jax.experimental.pallas import tpu_sc as plsc

_CH = 128
_M = 32
_N = _CH * _M
_K = 64
_NW = 32
_LG2N = 13
_G = 8


def _thresh(v):
    return jnp.where(v < 0, 0.0, v)


def _tc_scan_body(x_ref, c8_ref, f8_ref, tot_ref, xt_ref, c_ref):
    RB = x_ref.shape[0]

    for m in range(_M):
        xt_ref[:, m, :] = jnp.transpose(x_ref[:, m, :])

    carry = _thresh(xt_ref[0])
    c_ref[0] = carry
    for jj in range(1, _CH):
        carry = carry + _thresh(xt_ref[jj])
        c_ref[jj] = carry

    T = c_ref[_CH - 1]
    pm = jnp.zeros((RB,), jnp.float32)
    plist = []
    for m in range(_M):
        plist.append(pm)
        pm = pm + T[m]
    P = jnp.stack(plist, axis=0)

    c_ref[...] = c_ref[...] + P[None, :, :]
    tot_ref[...] = pm[None, :]

    for m in range(_M):
        c8_ref[:, m, :, :] = jnp.transpose(c_ref[:, m, :]).reshape(
            RB // _G, _G, _CH)
        f8_ref[:, m, :, :] = jnp.transpose(_thresh(xt_ref[:, m, :])).reshape(
            RB // _G, _G, _CH)


def _sc_search_body(rpw, c8_hbm, u_hbm, tot_hbm, f8_hbm, s_hbm, p_hbm,
                    cbuf, fbuf, ubuf, tbuf, sall, pall, semc, semf):
    ng = rpw // _G
    wid = lax.axis_index("s") * 2 + lax.axis_index("c")
    base = wid * rpw
    gbase = wid * ng

    pltpu.sync_copy(u_hbm.at[pl.ds(base * _K, rpw * _K)], ubuf)
    pltpu.sync_copy(tot_hbm.at[pl.ds(base, rpw)], tbuf)

    pltpu.async_copy(c8_hbm.at[gbase], cbuf.at[0], semc.at[0])

    def group_body(g, _):
        buf = lax.rem(g, 2)
        nbuf = 1 - buf
        @pl.when(g + 1 < ng)
        def _():
            pltpu.async_copy(c8_hbm.at[gbase + g + 1], cbuf.at[nbuf],
                             semc.at[nbuf])
        fcopy = pltpu.async_copy(f8_hbm.at[gbase + g], fbuf, semf)
        pltpu.make_async_copy(c8_hbm.at[gbase + g], cbuf.at[buf],
                              semc.at[buf]).wait()

        buf16 = jnp.full((16,), buf, jnp.int32)

        def row_body(rlo, _2):
            r = g * _G + rlo
            t = plsc.load_gather(tbuf, [jnp.full((16,), r, jnp.int32)])
            off = rlo * _CH
            for gk in range(_K // 16):
                uraw = ubuf[pl.ds(r * _K + gk * 16, 16)]
                u2 = uraw * t
                lo = jnp.zeros((16,), jnp.int32)
                hi = jnp.full((16,), _N, jnp.int32)
                for _step in range(_LG2N):
                    mid = jnp.minimum(jnp.right_shift(lo + hi, 1), _N - 1)
                    adr = ((mid >> 7) << 10) + off + (mid & 127)
                    v = plsc.load_gather(cbuf, [buf16, adr])
                    pred = v <= u2
                    lo = jnp.where(pred, mid + 1, lo)
                    hi = jnp.where(pred, hi, mid)
                s = jnp.minimum(lo, _N - 1)
                sall[pl.ds(r * _K + gk * 16, 16)] = s
            return 0

        lax.fori_loop(0, _G, row_body, 0, unroll=False)

        fcopy.wait()

        def prob_body(rlo, _2):
            r = g * _G + rlo
            off = rlo * _CH
            for gk in range(_K // 16):
                s = sall[pl.ds(r * _K + gk * 16, 16)]
                adr = ((s >> 7) << 10) + off + (s & 127)
                vals = plsc.load_gather(fbuf, [adr])
                pall[pl.ds(r * _K + gk * 16, 16)] = vals
            return 0

        lax.fori_loop(0, _G, prob_body, 0, unroll=False)
        return 0

    lax.fori_loop(0, ng, group_body, 0, unroll=False)

    pltpu.sync_copy(sall, s_hbm.at[pl.ds(base * _K, rpw * _K)])
    pltpu.sync_copy(pall, p_hbm.at[pl.ds(base * _K, rpw * _K)])


def _tc_sort_body(s_ref, p_ref, xs_ref, ys_ref):
    RB, K = s_ref.shape
    s = jnp.transpose(s_ref[...])
    p = jnp.transpose(p_ref[...])

    ki = lax.broadcasted_iota(jnp.int32, (K, 1), 0)
    rank = jnp.zeros(s.shape, jnp.int32)
    for kq in range(K):
        pq = p[kq][None, :]
        before = (pq > p) | ((pq == p) & (kq < ki))
        rank = rank + before.astype(jnp.int32)

    s_sorted = jnp.zeros(s.shape, jnp.int32)
    for kq in range(K):
        hit = rank[kq][None, :] == ki
        s_sorted = s_sorted + jnp.where(hit, s[kq][None, :], 0)

    xf = (s_sorted & 63).astype(jnp.float32)
    yf = (s_sorted >> 6).astype(jnp.float32)
    xs_ref[...] = (xf - 32.0) * 0.015625
    ys_ref[...] = (yf - 32.0) * 0.015625


def _run_tc_scan(x_rows):
    Rh = x_rows.shape[0]
    RB = 256 if Rh % 256 == 0 else Rh
    return pl.pallas_call(
        _tc_scan_body,
        grid=(Rh // RB,),
        in_specs=[pl.BlockSpec((RB, _M, _CH), lambda i: (i, 0, 0))],
        out_specs=[
            pl.BlockSpec((RB // _G, _M, _G, _CH), lambda i: (i, 0, 0, 0)),
            pl.BlockSpec((RB // _G, _M, _G, _CH), lambda i: (i, 0, 0, 0)),
            pl.BlockSpec((1, RB), lambda i: (0, i)),
        ],
        out_shape=[
            jax.ShapeDtypeStruct((Rh // _G, _M, _G, _CH), jnp.float32),
            jax.ShapeDtypeStruct((Rh // _G, _M, _G, _CH), jnp.float32),
            jax.ShapeDtypeStruct((1, Rh), jnp.float32),
        ],
        scratch_shapes=[
            pltpu.VMEM((_CH, _M, RB), jnp.float32),
            pltpu.VMEM((_CH, _M, RB), jnp.float32),
        ],
        compiler_params=pltpu.CompilerParams(
            dimension_semantics=("arbitrary",),
        ),
    )(x_rows)


def _run_sc(c8, f8, u_flat, tot_flat):
    Rh = tot_flat.shape[0]
    rpw = Rh // _NW
    mesh = plsc.VectorSubcoreMesh(core_axis_name="c", subcore_axis_name="s")
    sc = pl.kernel(
        functools.partial(_sc_search_body, rpw),
        out_type=[
            jax.ShapeDtypeStruct((Rh * _K,), jnp.int32),
            jax.ShapeDtypeStruct((Rh * _K,), jnp.float32),
        ],
        mesh=mesh,
        scratch_types=[
            pltpu.VMEM((2, _M * _G * _CH), jnp.float32),
            pltpu.VMEM((_M * _G * _CH,), jnp.float32),
            pltpu.VMEM((rpw * _K,), jnp.float32),
            pltpu.VMEM((rpw,), jnp.float32),
            pltpu.VMEM((rpw * _K,), jnp.int32),
            pltpu.VMEM((rpw * _K,), jnp.float32),
            pltpu.SemaphoreType.DMA((2,)),
            pltpu.SemaphoreType.DMA,
        ],
        compiler_params=pltpu.CompilerParams(needs_layout_passes=False),
    )
    return sc(
        c8.reshape(Rh // _G, _M * _G * _CH),
        u_flat,
        tot_flat,
        f8.reshape(Rh // _G, _M * _G * _CH),
    )


def _run_tc_sort(s2d, p2d):
    Rh = s2d.shape[0]
    RB = 256 if Rh % 256 == 0 else Rh
    return pl.pallas_call(
        _tc_sort_body,
        grid=(Rh // RB,),
        in_specs=[
            pl.BlockSpec((RB, _K), lambda i: (i, 0)),
            pl.BlockSpec((RB, _K), lambda i: (i, 0)),
        ],
        out_specs=[
            pl.BlockSpec((_K, RB), lambda i: (0, i)),
            pl.BlockSpec((_K, RB), lambda i: (0, i)),
        ],
        out_shape=[
            jax.ShapeDtypeStruct((_K, Rh), jnp.float32),
            jax.ShapeDtypeStruct((_K, Rh), jnp.float32),
        ],
        compiler_params=pltpu.CompilerParams(
            dimension_semantics=("arbitrary",),
        ),
    )(s2d, p2d)


_UCACHE = {}


def _u_table(R):
    if R not in _UCACHE:
        import numpy as _np
        with jax.ensure_compile_time_eval():
            _UCACHE[R] = _np.asarray(
                jax.random.uniform(jax.random.key(42), (R, _K),
                                   dtype=jnp.float32))
    return _UCACHE[R]


def kernel(heatmap, num_samples):
    b, j, w, h = heatmap.shape
    R = b * j

    u_raw = jnp.asarray(_u_table(R))
    x_all = heatmap.reshape(R, _M, _CH)

    splits = [(0, R)]

    xs_parts, ys_parts = [], []
    sc_outs = []
    for off, ln in splits:
        c8, f8, tot = _run_tc_scan(lax.slice_in_dim(x_all, off, off + ln, 1, 0))
        sc_outs.append(_run_sc(
            c8, f8,
            lax.slice_in_dim(u_raw.reshape(R * _K), off * _K,
                             (off + ln) * _K, 1, 0),
            tot.reshape(ln)))
    for (off, ln), (s_fl, p_fl) in zip(splits, sc_outs):
        xs_h, ys_h = _run_tc_sort(s_fl.reshape(ln, _K), p_fl.reshape(ln, _K))
        xs_parts.append(xs_h)
        ys_parts.append(ys_h)

    xs = jnp.concatenate(xs_parts, axis=1) if len(xs_parts) > 1 else xs_parts[0]
    ys = jnp.concatenate(ys_parts, axis=1) if len(ys_parts) > 1 else ys_parts[0]

    xn = xs.reshape(_K, b, j)
    yn = ys.reshape(_K, b, j)
    out = jnp.stack((xn, yn), axis=-1).transpose(1, 0, 2, 3).reshape(
        b, _K, 2 * j)
    return out


# speedup vs baseline: 2.3196x; 2.2880x over previous
import functools

import jax
import jax.numpy as jnp
from jax import lax
from jax.experimental import pallas as pl
from jax.experimental.pallas import tpu as pltpu
from jax.experimental.pallas import tpu_sc as plsc

_CH = 128
_M = 32
_N = _CH * _M
_K = 64
_NW = 32
_LG2N = 13
_G = 8


def _thresh(v):
    return jnp.where(v < 0, 0.0, v)


def _tc_scan_body(x_ref, c8_ref, f8_ref, tot_ref, xt_ref, c_ref):
    RB = x_ref.shape[-1]

    xmjr = x_ref[0].reshape(_M, _CH, RB)
    xt_ref[...] = jnp.swapaxes(xmjr, 0, 1)

    carry = _thresh(xt_ref[0])
    c_ref[0] = carry
    for jj in range(1, _CH):
        carry = carry + _thresh(xt_ref[jj])
        c_ref[jj] = carry

    T = c_ref[_CH - 1]
    pm = jnp.zeros((RB,), jnp.float32)
    plist = []
    for m in range(_M):
        plist.append(pm)
        pm = pm + T[m]
    P = jnp.stack(plist, axis=0)

    c_ref[...] = c_ref[...] + P[None, :, :]
    tot_ref[...] = pm[None, :]

    for m in range(_M):
        c8_ref[:, m, :, :] = jnp.transpose(c_ref[:, m, :]).reshape(
            RB // _G, _G, _CH)
        f8_ref[:, m, :, :] = jnp.transpose(_thresh(xt_ref[:, m, :])).reshape(
            RB // _G, _G, _CH)


def _sc_search_body(rpw, c8_hbm, u_hbm, tot_hbm, f8_hbm, s_hbm, p_hbm,
                    cbuf, fbuf, ubuf, tbuf, sall, pall, semc, semf):
    ng = rpw // _G
    wid = lax.axis_index("s") * 2 + lax.axis_index("c")
    base = wid * rpw
    gbase = wid * ng

    pltpu.sync_copy(u_hbm.at[pl.ds(base * _K, rpw * _K)], ubuf)
    pltpu.sync_copy(tot_hbm.at[pl.ds(base, rpw)], tbuf)

    pltpu.async_copy(c8_hbm.at[gbase], cbuf.at[0], semc.at[0])

    def group_body(g, _):
        buf = lax.rem(g, 2)
        nbuf = 1 - buf
        @pl.when(g + 1 < ng)
        def _():
            pltpu.async_copy(c8_hbm.at[gbase + g + 1], cbuf.at[nbuf],
                             semc.at[nbuf])
        fcopy = pltpu.async_copy(f8_hbm.at[gbase + g], fbuf, semf)
        pltpu.make_async_copy(c8_hbm.at[gbase + g], cbuf.at[buf],
                              semc.at[buf]).wait()

        buf16 = jnp.full((16,), buf, jnp.int32)

        def row_body(rlo, _2):
            r = g * _G + rlo
            t = plsc.load_gather(tbuf, [jnp.full((16,), r, jnp.int32)])
            rlo16 = jnp.full((16,), rlo, jnp.int32)
            for gk in range(_K // 16):
                uraw = ubuf[pl.ds(r * _K + gk * 16, 16)]
                u2 = uraw * t
                lo = jnp.zeros((16,), jnp.int32)
                hi = jnp.full((16,), _N, jnp.int32)
                for _step in range(_LG2N):
                    mid = jnp.minimum(jnp.right_shift(lo + hi, 1), _N - 1)
                    v = plsc.load_gather(
                        cbuf, [buf16, mid >> 7, rlo16, mid & 127])
                    pred = v <= u2
                    lo = jnp.where(pred, mid + 1, lo)
                    hi = jnp.where(pred, hi, mid)
                s = jnp.minimum(lo, _N - 1)
                sall[pl.ds(r * _K + gk * 16, 16)] = s
            return 0

        lax.fori_loop(0, _G, row_body, 0, unroll=False)

        fcopy.wait()

        def prob_body(rlo, _2):
            r = g * _G + rlo
            rlo16 = jnp.full((16,), rlo, jnp.int32)
            for gk in range(_K // 16):
                s = sall[pl.ds(r * _K + gk * 16, 16)]
                vals = plsc.load_gather(fbuf, [s >> 7, rlo16, s & 127])
                pall[pl.ds(r * _K + gk * 16, 16)] = vals
            return 0

        lax.fori_loop(0, _G, prob_body, 0, unroll=False)
        return 0

    lax.fori_loop(0, ng, group_body, 0, unroll=False)

    pltpu.sync_copy(sall, s_hbm.at[pl.ds(base * _K, rpw * _K)])
    pltpu.sync_copy(pall, p_hbm.at[pl.ds(base * _K, rpw * _K)])


def _tc_sort_body(s_ref, p_ref, xs_ref, ys_ref):
    RB, K = s_ref.shape
    s = jnp.transpose(s_ref[...])
    p = jnp.transpose(p_ref[...])

    ki = lax.broadcasted_iota(jnp.int32, (K, 1), 0)
    rank = jnp.zeros(s.shape, jnp.int32)
    for kq in range(K):
        pq = p[kq][None, :]
        before = (pq > p) | ((pq == p) & (kq < ki))
        rank = rank + before.astype(jnp.int32)

    s_sorted = jnp.zeros(s.shape, jnp.int32)
    for kq in range(K):
        hit = rank[kq][None, :] == ki
        s_sorted = s_sorted + jnp.where(hit, s[kq][None, :], 0)

    xf = (s_sorted & 63).astype(jnp.float32)
    yf = (s_sorted >> 6).astype(jnp.float32)
    xs_ref[...] = (xf - 32.0) * 0.015625
    ys_ref[...] = (yf - 32.0) * 0.015625


def _run_tc_scan(ht):
    J, W, H, B = ht.shape
    Rh = J * B
    RB = B
    return pl.pallas_call(
        _tc_scan_body,
        grid=(J,),
        in_specs=[pl.BlockSpec((1, W, H, B), lambda i: (i, 0, 0, 0))],
        out_specs=[
            pl.BlockSpec((RB // _G, _M, _G, _CH), lambda i: (i, 0, 0, 0)),
            pl.BlockSpec((RB // _G, _M, _G, _CH), lambda i: (i, 0, 0, 0)),
            pl.BlockSpec((1, RB), lambda i: (0, i)),
        ],
        out_shape=[
            jax.ShapeDtypeStruct((Rh // _G, _M, _G, _CH), jnp.float32),
            jax.ShapeDtypeStruct((Rh // _G, _M, _G, _CH), jnp.float32),
            jax.ShapeDtypeStruct((1, Rh), jnp.float32),
        ],
        scratch_shapes=[
            pltpu.VMEM((_CH, _M, RB), jnp.float32),
            pltpu.VMEM((_CH, _M, RB), jnp.float32),
        ],
        compiler_params=pltpu.CompilerParams(
            dimension_semantics=("arbitrary",),
        ),
    )(ht)


def _run_sc(c8, f8, u_flat, tot_flat):
    Rh = tot_flat.shape[0]
    rpw = Rh // _NW
    mesh = plsc.VectorSubcoreMesh(core_axis_name="c", subcore_axis_name="s")
    sc = pl.kernel(
        functools.partial(_sc_search_body, rpw),
        out_type=[
            jax.ShapeDtypeStruct((Rh * _K,), jnp.int32),
            jax.ShapeDtypeStruct((Rh * _K,), jnp.float32),
        ],
        mesh=mesh,
        scratch_types=[
            pltpu.VMEM((2, _M, _G, _CH), jnp.float32),
            pltpu.VMEM((_M, _G, _CH), jnp.float32),
            pltpu.VMEM((rpw * _K,), jnp.float32),
            pltpu.VMEM((rpw,), jnp.float32),
            pltpu.VMEM((rpw * _K,), jnp.int32),
            pltpu.VMEM((rpw * _K,), jnp.float32),
            pltpu.SemaphoreType.DMA((2,)),
            pltpu.SemaphoreType.DMA,
        ],
        compiler_params=pltpu.CompilerParams(needs_layout_passes=False),
    )
    return sc(c8, u_flat, tot_flat, f8)


def _run_tc_sort(s2d, p2d):
    Rh = s2d.shape[0]
    RB = 256 if Rh % 256 == 0 else Rh
    return pl.pallas_call(
        _tc_sort_body,
        grid=(Rh // RB,),
        in_specs=[
            pl.BlockSpec((RB, _K), lambda i: (i, 0)),
            pl.BlockSpec((RB, _K), lambda i: (i, 0)),
        ],
        out_specs=[
            pl.BlockSpec((_K, RB), lambda i: (0, i)),
            pl.BlockSpec((_K, RB), lambda i: (0, i)),
        ],
        out_shape=[
            jax.ShapeDtypeStruct((_K, Rh), jnp.float32),
            jax.ShapeDtypeStruct((_K, Rh), jnp.float32),
        ],
        compiler_params=pltpu.CompilerParams(
            dimension_semantics=("arbitrary",),
        ),
    )(s2d, p2d)


def kernel(heatmap, num_samples):
    b, j, w, h = heatmap.shape
    R = b * j

    ht = jnp.transpose(heatmap, (1, 2, 3, 0))
    u_perm = jax.random.uniform(
        jax.random.key(42), (R, _K), dtype=jnp.float32
    ).reshape(b, j, _K).transpose(1, 0, 2).reshape(R * _K)

    c8, f8, tot = _run_tc_scan(ht)
    s_fl, p_fl = _run_sc(c8, f8, u_perm, tot.reshape(R))
    xs, ys = _run_tc_sort(s_fl.reshape(R, _K), p_fl.reshape(R, _K))

    xn = xs.reshape(_K, j, b)
    yn = ys.reshape(_K, j, b)
    out = jnp.stack((xn, yn), axis=-1).transpose(2, 0, 1, 3).reshape(
        b, _K, 2 * j)
    return out
